# Initial kernel scaffold; baseline (speedup 1.0000x reference)
import jax
import jax.numpy as jnp
from jax.experimental import pallas as pl


def kernel(system_states_trajectory, eligibility_traces_trajectory, activity_traces_trajectory, inverse_state_norms_trajectory, variational_gradient_trajectory, weight_values, weight_rows, weight_cols, trophic_support_map, activity_bias):
    raise NotImplementedError("write your pallas kernel here")


# speedup vs baseline: 36.7976x; 36.7976x over previous
import functools

import jax
import jax.numpy as jnp
from jax import lax
from jax.experimental import pallas as pl
from jax.experimental.pallas import tpu as pltpu
from jax.experimental.pallas import tpu_sc as plsc

NUM_BLOCKS = 1024
NPB = 4
B = 16
T = 32
NNZ = 32768
N = NUM_BLOCKS * NPB
HEBB_LR = 0.01
OJA_DECAY = 0.001
MAX_NORM = 1.0
DELTA_MAX_NORM = 0.1
EPS = 1e-6

NW = 32
EPW = NNZ // NW
BLK = NPB * NPB
CHUNK = 128
NCHUNK = EPW // CHUNK
NGRP = CHUNK // 16


def _prep_body(s_ref, e_ref, v_ref, inv_ref, q_ref, sd_ref, h_ref):
    s = s_ref[0]
    e = e_ref[0]
    v = v_ref[0]
    inv = inv_ref[0]
    q_ref[0] = (v * (1.0 - s * s) + 0.5 * e) * (1.0 / B)
    sd_ref[0] = s * inv
    h_ref[0, 0] = jnp.sum(s * s, axis=0) * (OJA_DECAY / B)


def _prep(s_traj, e_traj, v_traj, inv_traj):
    return pl.pallas_call(
        _prep_body,
        grid=(T,),
        in_specs=[
            pl.BlockSpec((1, B, N), lambda t: (t, 0, 0)),
            pl.BlockSpec((1, B, N), lambda t: (t, 0, 0)),
            pl.BlockSpec((1, B, N), lambda t: (t, 0, 0)),
            pl.BlockSpec((1, B, 1), lambda t: (t, 0, 0)),
        ],
        out_specs=[
            pl.BlockSpec((1, B, N), lambda t: (t, 0, 0)),
            pl.BlockSpec((1, B, N), lambda t: (t, 0, 0)),
            pl.BlockSpec((1, 1, N), lambda t: (t, 0, 0)),
        ],
        out_shape=[
            jax.ShapeDtypeStruct((T, B, N), jnp.float32),
            jax.ShapeDtypeStruct((T, B, N), jnp.float32),
            jax.ShapeDtypeStruct((T, 1, N), jnp.float32),
        ],
    )(s_traj, e_traj, v_traj, inv_traj)


def _rsqrt(x):
    xi = lax.bitcast_convert_type(x, jnp.int32)
    yi = jnp.int32(0x5F3759DF) - lax.shift_right_logical(xi, 1)
    y = lax.bitcast_convert_type(yi, jnp.float32)
    xh = 0.5 * x
    for _ in range(3):
        y = y * (1.5 - xh * y * y)
    return y


_MESH = plsc.VectorSubcoreMesh(core_axis_name="c", subcore_axis_name="s")


@functools.partial(
    pl.kernel,
    mesh=_MESH,
    compiler_params=pltpu.CompilerParams(needs_layout_passes=False,
                                         use_tc_tiling_on_sc=False),
    out_type=jax.ShapeDtypeStruct((NW * BLK * EPW,), jnp.float32),
    scratch_types=[
        pltpu.VMEM((EPW,), jnp.int32),
        pltpu.VMEM((EPW,), jnp.int32),
        pltpu.VMEM((BLK * EPW,), jnp.float32),
        pltpu.VMEM((N,), jnp.float32),
        pltpu.VMEM((NUM_BLOCKS,), jnp.float32),
        pltpu.VMEM((CHUNK,), jnp.int32),
        pltpu.VMEM((CHUNK,), jnp.int32),
        pltpu.VMEM((CHUNK, B * NPB), jnp.float32),
        pltpu.VMEM((CHUNK, B * NPB), jnp.float32),
        pltpu.SemaphoreType.DMA,
    ],
)
def _sc_update(q_hbm, s_hbm, h_hbm, rows_hbm, cols_hbm, w_hbm, w_out,
               rows_v, cols_v, w_v, h_v, g_v, idxr_v, idxc_v, qr_v, sr_v,
               sem):
    wid = lax.axis_index("s") * 2 + lax.axis_index("c")
    ebase = wid * EPW
    wbase = wid * (BLK * EPW)
    pltpu.sync_copy(rows_hbm.at[pl.ds(ebase, EPW)], rows_v)
    pltpu.sync_copy(cols_hbm.at[pl.ds(ebase, EPW)], cols_v)
    pltpu.sync_copy(w_hbm.at[pl.ds(wbase, BLK * EPW)], w_v)

    lanes = lax.iota(jnp.int32, 16)

    def t_body(t, carry):
        pltpu.sync_copy(h_hbm.at[pl.ds(t * N, N)], h_v)

        def g_body(j, c2):
            base = j * 16
            bi = (lanes + base) * NPB
            acc = plsc.load_gather(h_v, [bi])
            for cc in range(1, NPB):
                acc = acc + plsc.load_gather(h_v, [bi + cc])
            g_v[pl.ds(base, 16)] = acc
            return c2

        lax.fori_loop(0, NUM_BLOCKS // 16, g_body, 0)

        toff = t * NUM_BLOCKS

        def chunk_body(c, c2):
            cbase = c * CHUNK
            for j in range(CHUNK // 16):
                rseg = rows_v[pl.ds(cbase + j * 16, 16)]
                cseg = cols_v[pl.ds(cbase + j * 16, 16)]
                idxr_v[pl.ds(j * 16, 16)] = rseg + toff
                idxc_v[pl.ds(j * 16, 16)] = cseg + toff
            cq = pltpu.async_copy(q_hbm.at[idxr_v], qr_v, sem)
            cs = pltpu.async_copy(s_hbm.at[idxc_v], sr_v, sem)
            cq.wait()
            cs.wait()

            def grp_body(gi, c3):
                e0 = gi * 16
                elanes = lanes + e0
                acc = [jnp.zeros((16,), jnp.float32) for _ in range(BLK)]
                for b in range(B):
                    qa = [
                        plsc.load_gather(
                            qr_v,
                            [elanes, jnp.full((16,), b * NPB + i, jnp.int32)])
                        for i in range(NPB)
                    ]
                    sb = [
                        plsc.load_gather(
                            sr_v,
                            [elanes, jnp.full((16,), b * NPB + j2, jnp.int32)])
                        for j2 in range(NPB)
                    ]
                    for i in range(NPB):
                        for j2 in range(NPB):
                            acc[i * NPB + j2] = acc[i * NPB + j2] + qa[i] * sb[j2]
                off = cbase + e0
                rv = rows_v[pl.ds(off, 16)]
                gq = plsc.load_gather(g_v, [rv])
                wv = [w_v[pl.ds(ij * EPW + off, 16)] for ij in range(BLK)]
                d = [acc[ij] - gq * wv[ij] for ij in range(BLK)]
                ssd = d[0] * d[0]
                for ij in range(1, BLK):
                    ssd = ssd + d[ij] * d[ij]
                scd = jnp.minimum(1.0, DELTA_MAX_NORM * _rsqrt(ssd + EPS))
                step = HEBB_LR * scd
                wn = [wv[ij] + step * d[ij] for ij in range(BLK)]
                ssw = wn[0] * wn[0]
                for ij in range(1, BLK):
                    ssw = ssw + wn[ij] * wn[ij]
                scw = jnp.minimum(1.0, MAX_NORM * _rsqrt(ssw + EPS))
                for ij in range(BLK):
                    w_v[pl.ds(ij * EPW + off, 16)] = wn[ij] * scw
                return c3

            lax.fori_loop(0, NGRP, grp_body, 0)
            return c2

        lax.fori_loop(0, NCHUNK, chunk_body, 0)
        return carry

    lax.fori_loop(0, T, t_body, 0)
    pltpu.sync_copy(w_v, w_out.at[pl.ds(wbase, BLK * EPW)])


def kernel(system_states_trajectory, eligibility_traces_trajectory,
           activity_traces_trajectory, inverse_state_norms_trajectory,
           variational_gradient_trajectory, weight_values, weight_rows,
           weight_cols, trophic_support_map, activity_bias):
    del activity_traces_trajectory, trophic_support_map, activity_bias
    qd, sd, hd = _prep(system_states_trajectory,
                       eligibility_traces_trajectory,
                       variational_gradient_trajectory,
                       inverse_state_norms_trajectory)
    q_tbl = qd.reshape(T, B, NUM_BLOCKS, NPB).transpose(0, 2, 1, 3).reshape(
        T * NUM_BLOCKS, B * NPB)
    s_tbl = sd.reshape(T, B, NUM_BLOCKS, NPB).transpose(0, 2, 1, 3).reshape(
        T * NUM_BLOCKS, B * NPB)
    h_flat = hd.reshape(T * N)
    w_flat = weight_values.reshape(NW, EPW, BLK).transpose(0, 2, 1).reshape(-1)
    w_out = _sc_update(q_tbl, s_tbl, h_flat, weight_rows, weight_cols, w_flat)
    return w_out.reshape(NW, BLK, EPW).transpose(0, 2, 1).reshape(
        NNZ, NPB, NPB)


# speedup vs baseline: 42.4787x; 1.1544x over previous
import functools

import jax
import jax.numpy as jnp
from jax import lax
from jax.experimental import pallas as pl
from jax.experimental.pallas import tpu as pltpu
from jax.experimental.pallas import tpu_sc as plsc

NUM_BLOCKS = 1024
NPB = 4
B = 16
T = 32
NNZ = 32768
N = NUM_BLOCKS * NPB
HEBB_LR = 0.01
OJA_DECAY = 0.001
MAX_NORM = 1.0
DELTA_MAX_NORM = 0.1
EPS = 1e-6

NW = 32
EPW = NNZ // NW
BLK = NPB * NPB
CHUNK = 128
NCHUNK = EPW // CHUNK
NGRP = CHUNK // 16


def _prep_body(s_ref, e_ref, v_ref, inv_ref, q_ref, sd_ref, h_ref):
    s = s_ref[0]
    e = e_ref[0]
    v = v_ref[0]
    inv = inv_ref[0]
    q_ref[0] = (v * (1.0 - s * s) + 0.5 * e) * (1.0 / B)
    sd_ref[0] = s * inv
    h_ref[0, 0] = jnp.sum(s * s, axis=0) * (OJA_DECAY / B)


def _prep(s_traj, e_traj, v_traj, inv_traj):
    return pl.pallas_call(
        _prep_body,
        grid=(T,),
        in_specs=[
            pl.BlockSpec((1, B, N), lambda t: (t, 0, 0)),
            pl.BlockSpec((1, B, N), lambda t: (t, 0, 0)),
            pl.BlockSpec((1, B, N), lambda t: (t, 0, 0)),
            pl.BlockSpec((1, B, 1), lambda t: (t, 0, 0)),
        ],
        out_specs=[
            pl.BlockSpec((1, B, N), lambda t: (t, 0, 0)),
            pl.BlockSpec((1, B, N), lambda t: (t, 0, 0)),
            pl.BlockSpec((1, 1, N), lambda t: (t, 0, 0)),
        ],
        out_shape=[
            jax.ShapeDtypeStruct((T, B, N), jnp.float32),
            jax.ShapeDtypeStruct((T, B, N), jnp.float32),
            jax.ShapeDtypeStruct((T, 1, N), jnp.float32),
        ],
    )(s_traj, e_traj, v_traj, inv_traj)


def _rsqrt(x):
    xi = lax.bitcast_convert_type(x, jnp.int32)
    yi = jnp.int32(0x5F3759DF) - lax.shift_right_logical(xi, 1)
    y = lax.bitcast_convert_type(yi, jnp.float32)
    xh = 0.5 * x
    for _ in range(3):
        y = y * (1.5 - xh * y * y)
    return y


_MESH = plsc.VectorSubcoreMesh(core_axis_name="c", subcore_axis_name="s")


@functools.partial(
    pl.kernel,
    mesh=_MESH,
    compiler_params=pltpu.CompilerParams(needs_layout_passes=False,
                                         use_tc_tiling_on_sc=False),
    out_type=jax.ShapeDtypeStruct((NW * BLK * EPW,), jnp.float32),
    scratch_types=[
        pltpu.VMEM((EPW,), jnp.int32),
        pltpu.VMEM((EPW,), jnp.int32),
        pltpu.VMEM((BLK * EPW,), jnp.float32),
        pltpu.VMEM((N,), jnp.float32),
        pltpu.VMEM((NUM_BLOCKS,), jnp.float32),
        pltpu.VMEM((CHUNK,), jnp.int32),
        pltpu.VMEM((CHUNK,), jnp.int32),
        pltpu.VMEM((CHUNK, B * NPB), jnp.float32),
        pltpu.VMEM((CHUNK, B * NPB), jnp.float32),
        pltpu.SemaphoreType.DMA,
        pltpu.VMEM((CHUNK,), jnp.int32),
        pltpu.VMEM((CHUNK,), jnp.int32),
        pltpu.VMEM((CHUNK, B * NPB), jnp.float32),
        pltpu.VMEM((CHUNK, B * NPB), jnp.float32),
        pltpu.SemaphoreType.DMA,
    ],
)
def _sc_update(q_hbm, s_hbm, h_hbm, rows_hbm, cols_hbm, w_hbm, w_out,
               rows_v, cols_v, w_v, h_v, g_v,
               idxr0, idxc0, qr0, sr0, sem0,
               idxr1, idxc1, qr1, sr1, sem1):
    wid = lax.axis_index("s") * 2 + lax.axis_index("c")
    ebase = wid * EPW
    wbase = wid * (BLK * EPW)
    pltpu.sync_copy(rows_hbm.at[pl.ds(ebase, EPW)], rows_v)
    pltpu.sync_copy(cols_hbm.at[pl.ds(ebase, EPW)], cols_v)
    pltpu.sync_copy(w_hbm.at[pl.ds(wbase, BLK * EPW)], w_v)

    lanes = lax.iota(jnp.int32, 16)
    nsteps = T * NCHUNK

    def fire(s, idxr, idxc, qr, sr, sem):
        s = jnp.minimum(s, nsteps - 1)
        toff = (s // NCHUNK) * NUM_BLOCKS
        cbase = (s % NCHUNK) * CHUNK
        for j in range(CHUNK // 16):
            rseg = rows_v[pl.ds(cbase + j * 16, 16)]
            cseg = cols_v[pl.ds(cbase + j * 16, 16)]
            idxr[pl.ds(j * 16, 16)] = rseg + toff
            idxc[pl.ds(j * 16, 16)] = cseg + toff
        pltpu.async_copy(q_hbm.at[idxr], qr, sem)
        pltpu.async_copy(s_hbm.at[idxc], sr, sem)

    def wait_pair(idxr, idxc, qr, sr, sem):
        pltpu.make_async_copy(q_hbm.at[idxr], qr, sem).wait()
        pltpu.make_async_copy(s_hbm.at[idxc], sr, sem).wait()

    def update_g(t):
        pltpu.sync_copy(h_hbm.at[pl.ds(t * N, N)], h_v)

        def g_body(j, c2):
            base = j * 16
            bi = (lanes + base) * NPB
            acc = plsc.load_gather(h_v, [bi])
            for cc in range(1, NPB):
                acc = acc + plsc.load_gather(h_v, [bi + cc])
            g_v[pl.ds(base, 16)] = acc
            return c2

        lax.fori_loop(0, NUM_BLOCKS // 16, g_body, 0)

    def consume(s, idxr, idxc, qr, sr, sem):
        wait_pair(idxr, idxc, qr, sr, sem)
        cbase = (s % NCHUNK) * CHUNK

        def grp_body(gi, c3):
            e0 = gi * 16
            elanes = lanes + e0
            acc = [jnp.zeros((16,), jnp.float32) for _ in range(BLK)]
            for b in range(B):
                qa = [
                    plsc.load_gather(
                        qr, [elanes, jnp.full((16,), b * NPB + i, jnp.int32)])
                    for i in range(NPB)
                ]
                sb = [
                    plsc.load_gather(
                        sr, [elanes, jnp.full((16,), b * NPB + j2, jnp.int32)])
                    for j2 in range(NPB)
                ]
                for i in range(NPB):
                    for j2 in range(NPB):
                        acc[i * NPB + j2] = acc[i * NPB + j2] + qa[i] * sb[j2]
            off = cbase + e0
            rv = rows_v[pl.ds(off, 16)]
            gq = plsc.load_gather(g_v, [rv])
            wv = [w_v[pl.ds(ij * EPW + off, 16)] for ij in range(BLK)]
            d = [acc[ij] - gq * wv[ij] for ij in range(BLK)]
            ssd = d[0] * d[0]
            for ij in range(1, BLK):
                ssd = ssd + d[ij] * d[ij]
            scd = jnp.minimum(1.0, DELTA_MAX_NORM * _rsqrt(ssd + EPS))
            step = HEBB_LR * scd
            wn = [wv[ij] + step * d[ij] for ij in range(BLK)]
            ssw = wn[0] * wn[0]
            for ij in range(1, BLK):
                ssw = ssw + wn[ij] * wn[ij]
            scw = jnp.minimum(1.0, MAX_NORM * _rsqrt(ssw + EPS))
            for ij in range(BLK):
                w_v[pl.ds(ij * EPW + off, 16)] = wn[ij] * scw
            return c3

        lax.fori_loop(0, NGRP, grp_body, 0)

    buf0 = (idxr0, idxc0, qr0, sr0, sem0)
    buf1 = (idxr1, idxc1, qr1, sr1, sem1)

    fire(0, *buf0)

    def pair_body(i, carry):
        s = i * 2

        @pl.when(s % NCHUNK == 0)
        def _():
            update_g(s // NCHUNK)

        fire(s + 1, *buf1)
        consume(s, *buf0)
        fire(s + 2, *buf0)
        consume(s + 1, *buf1)
        return carry

    lax.fori_loop(0, nsteps // 2, pair_body, 0)
    wait_pair(*buf0)
    pltpu.sync_copy(w_v, w_out.at[pl.ds(wbase, BLK * EPW)])


def kernel(system_states_trajectory, eligibility_traces_trajectory,
           activity_traces_trajectory, inverse_state_norms_trajectory,
           variational_gradient_trajectory, weight_values, weight_rows,
           weight_cols, trophic_support_map, activity_bias):
    del activity_traces_trajectory, trophic_support_map, activity_bias
    qd, sd, hd = _prep(system_states_trajectory,
                       eligibility_traces_trajectory,
                       variational_gradient_trajectory,
                       inverse_state_norms_trajectory)
    q_tbl = qd.reshape(T, B, NUM_BLOCKS, NPB).transpose(0, 2, 1, 3).reshape(
        T * NUM_BLOCKS, B * NPB)
    s_tbl = sd.reshape(T, B, NUM_BLOCKS, NPB).transpose(0, 2, 1, 3).reshape(
        T * NUM_BLOCKS, B * NPB)
    h_flat = hd.reshape(T * N)
    w_flat = weight_values.reshape(NW, EPW, BLK).transpose(0, 2, 1).reshape(-1)
    w_out = _sc_update(q_tbl, s_tbl, h_flat, weight_rows, weight_cols, w_flat)
    return w_out.reshape(NW, BLK, EPW).transpose(0, 2, 1).reshape(
        NNZ, NPB, NPB)


# speedup vs baseline: 43.0143x; 1.0126x over previous
import functools

import jax
import jax.numpy as jnp
from jax import lax
from jax.experimental import pallas as pl
from jax.experimental.pallas import tpu as pltpu
from jax.experimental.pallas import tpu_sc as plsc

NUM_BLOCKS = 1024
NPB = 4
B = 16
T = 32
NNZ = 32768
N = NUM_BLOCKS * NPB
HEBB_LR = 0.01
OJA_DECAY = 0.001
MAX_NORM = 1.0
DELTA_MAX_NORM = 0.1
EPS = 1e-6

NW = 32
EPW = NNZ // NW
BLK = NPB * NPB
CHUNK = 128
NCHUNK = EPW // CHUNK
NGRP = CHUNK // 16


def _prep_body(s_ref, e_ref, v_ref, inv_ref, q_ref, sd_ref, h_ref):
    s = s_ref[0]
    e = e_ref[0]
    v = v_ref[0]
    inv = inv_ref[0]
    q_ref[0] = (v * (1.0 - s * s) + 0.5 * e) * (1.0 / B)
    sd_ref[0] = s * inv
    h_ref[0, 0] = jnp.sum(s * s, axis=0) * (OJA_DECAY / B)


def _prep(s_traj, e_traj, v_traj, inv_traj):
    return pl.pallas_call(
        _prep_body,
        grid=(T,),
        in_specs=[
            pl.BlockSpec((1, B, N), lambda t: (t, 0, 0)),
            pl.BlockSpec((1, B, N), lambda t: (t, 0, 0)),
            pl.BlockSpec((1, B, N), lambda t: (t, 0, 0)),
            pl.BlockSpec((1, B, 1), lambda t: (t, 0, 0)),
        ],
        out_specs=[
            pl.BlockSpec((1, B, N), lambda t: (t, 0, 0)),
            pl.BlockSpec((1, B, N), lambda t: (t, 0, 0)),
            pl.BlockSpec((1, 1, N), lambda t: (t, 0, 0)),
        ],
        out_shape=[
            jax.ShapeDtypeStruct((T, B, N), jnp.float32),
            jax.ShapeDtypeStruct((T, B, N), jnp.float32),
            jax.ShapeDtypeStruct((T, 1, N), jnp.float32),
        ],
    )(s_traj, e_traj, v_traj, inv_traj)


def _rsqrt(x):
    xi = lax.bitcast_convert_type(x, jnp.int32)
    yi = jnp.int32(0x5F3759DF) - lax.shift_right_logical(xi, 1)
    y = lax.bitcast_convert_type(yi, jnp.float32)
    xh = 0.5 * x
    for _ in range(3):
        y = y * (1.5 - xh * y * y)
    return y


_DO_DMA = True
_DO_COMPUTE = True

_MESH = plsc.VectorSubcoreMesh(core_axis_name="c", subcore_axis_name="s")


@functools.partial(
    pl.kernel,
    mesh=_MESH,
    compiler_params=pltpu.CompilerParams(needs_layout_passes=False,
                                         use_tc_tiling_on_sc=False),
    out_type=jax.ShapeDtypeStruct((NW * BLK * EPW,), jnp.float32),
    scratch_types=[
        pltpu.VMEM((EPW,), jnp.int32),
        pltpu.VMEM((EPW,), jnp.int32),
        pltpu.VMEM((BLK * EPW,), jnp.float32),
        pltpu.VMEM((N,), jnp.float32),
        pltpu.VMEM((NUM_BLOCKS,), jnp.float32),
        pltpu.VMEM((CHUNK,), jnp.int32),
        pltpu.VMEM((CHUNK,), jnp.int32),
        pltpu.VMEM((CHUNK, B * NPB), jnp.float32),
        pltpu.VMEM((CHUNK, B * NPB), jnp.float32),
        pltpu.SemaphoreType.DMA,
        pltpu.VMEM((CHUNK,), jnp.int32),
        pltpu.VMEM((CHUNK,), jnp.int32),
        pltpu.VMEM((CHUNK, B * NPB), jnp.float32),
        pltpu.VMEM((CHUNK, B * NPB), jnp.float32),
        pltpu.SemaphoreType.DMA,
    ],
)
def _sc_update(q_hbm, s_hbm, h_hbm, rows_hbm, cols_hbm, w_hbm, w_out,
               rows_v, cols_v, w_v, h_v, g_v,
               idxr0, idxc0, qr0, sr0, sem0,
               idxr1, idxc1, qr1, sr1, sem1):
    wid = lax.axis_index("s") * 2 + lax.axis_index("c")
    ebase = wid * EPW
    wbase = wid * (BLK * EPW)
    pltpu.sync_copy(rows_hbm.at[pl.ds(ebase, EPW)], rows_v)
    pltpu.sync_copy(cols_hbm.at[pl.ds(ebase, EPW)], cols_v)
    pltpu.sync_copy(w_hbm.at[pl.ds(wbase, BLK * EPW)], w_v)

    lanes = lax.iota(jnp.int32, 16)
    nsteps = T * NCHUNK

    def fire(s, idxr, idxc, qr, sr, sem):
        s = jnp.minimum(s, nsteps - 1)
        toff = (s // NCHUNK) * NUM_BLOCKS
        cbase = (s % NCHUNK) * CHUNK
        for j in range(CHUNK // 16):
            rseg = rows_v[pl.ds(cbase + j * 16, 16)]
            cseg = cols_v[pl.ds(cbase + j * 16, 16)]
            idxr[pl.ds(j * 16, 16)] = rseg + toff
            idxc[pl.ds(j * 16, 16)] = cseg + toff
        if _DO_DMA:
            pltpu.async_copy(q_hbm.at[idxr], qr, sem)
            pltpu.async_copy(s_hbm.at[idxc], sr, sem)

    def wait_pair(idxr, idxc, qr, sr, sem):
        if _DO_DMA:
            pltpu.make_async_copy(q_hbm.at[idxr], qr, sem).wait()
            pltpu.make_async_copy(s_hbm.at[idxc], sr, sem).wait()

    def update_g(t):
        pltpu.sync_copy(h_hbm.at[pl.ds(t * N, N)], h_v)

        @plsc.parallel_loop(0, NUM_BLOCKS // 16, unroll=2)
        def g_body(j):
            base = j * 16
            bi = (lanes + base) * NPB
            acc = plsc.load_gather(h_v, [bi])
            for cc in range(1, NPB):
                acc = acc + plsc.load_gather(h_v, [bi + cc])
            g_v[pl.ds(base, 16)] = acc

    def consume(s, idxr, idxc, qr, sr, sem):
        wait_pair(idxr, idxc, qr, sr, sem)
        cbase = (s % NCHUNK) * CHUNK

        def grp_body(gi):
            e0 = gi * 16
            elanes = lanes + e0
            acc = [jnp.zeros((16,), jnp.float32) for _ in range(BLK)]
            for b in range(B):
                qa = [
                    plsc.load_gather(
                        qr, [elanes, jnp.full((16,), b * NPB + i, jnp.int32)])
                    for i in range(NPB)
                ]
                sb = [
                    plsc.load_gather(
                        sr, [elanes, jnp.full((16,), b * NPB + j2, jnp.int32)])
                    for j2 in range(NPB)
                ]
                for i in range(NPB):
                    for j2 in range(NPB):
                        acc[i * NPB + j2] = acc[i * NPB + j2] + qa[i] * sb[j2]
            off = cbase + e0
            rv = rows_v[pl.ds(off, 16)]
            gq = plsc.load_gather(g_v, [rv])
            wv = [w_v[pl.ds(ij * EPW + off, 16)] for ij in range(BLK)]
            d = [acc[ij] - gq * wv[ij] for ij in range(BLK)]
            ssd = d[0] * d[0]
            for ij in range(1, BLK):
                ssd = ssd + d[ij] * d[ij]
            scd = jnp.minimum(1.0, DELTA_MAX_NORM * _rsqrt(ssd + EPS))
            step = HEBB_LR * scd
            wn = [wv[ij] + step * d[ij] for ij in range(BLK)]
            ssw = wn[0] * wn[0]
            for ij in range(1, BLK):
                ssw = ssw + wn[ij] * wn[ij]
            scw = jnp.minimum(1.0, MAX_NORM * _rsqrt(ssw + EPS))
            for ij in range(BLK):
                w_v[pl.ds(ij * EPW + off, 16)] = wn[ij] * scw

        if _DO_COMPUTE:
            plsc.parallel_loop(0, NGRP, unroll=2)(grp_body)

    buf0 = (idxr0, idxc0, qr0, sr0, sem0)
    buf1 = (idxr1, idxc1, qr1, sr1, sem1)

    fire(0, *buf0)

    def pair_body(i, carry):
        s = i * 2

        @pl.when(s % NCHUNK == 0)
        def _():
            update_g(s // NCHUNK)

        fire(s + 1, *buf1)
        consume(s, *buf0)
        fire(s + 2, *buf0)
        consume(s + 1, *buf1)
        return carry

    lax.fori_loop(0, nsteps // 2, pair_body, 0)
    wait_pair(*buf0)
    pltpu.sync_copy(w_v, w_out.at[pl.ds(wbase, BLK * EPW)])


def kernel(system_states_trajectory, eligibility_traces_trajectory,
           activity_traces_trajectory, inverse_state_norms_trajectory,
           variational_gradient_trajectory, weight_values, weight_rows,
           weight_cols, trophic_support_map, activity_bias):
    del activity_traces_trajectory, trophic_support_map, activity_bias
    qd, sd, hd = _prep(system_states_trajectory,
                       eligibility_traces_trajectory,
                       variational_gradient_trajectory,
                       inverse_state_norms_trajectory)
    q_tbl = qd.reshape(T, B, NUM_BLOCKS, NPB).transpose(0, 2, 1, 3).reshape(
        T * NUM_BLOCKS, B * NPB)
    s_tbl = sd.reshape(T, B, NUM_BLOCKS, NPB).transpose(0, 2, 1, 3).reshape(
        T * NUM_BLOCKS, B * NPB)
    h_flat = hd.reshape(T * N)
    w_flat = weight_values.reshape(NW, EPW, BLK).transpose(0, 2, 1).reshape(-1)
    w_out = _sc_update(q_tbl, s_tbl, h_flat, weight_rows, weight_cols, w_flat)
    return w_out.reshape(NW, BLK, EPW).transpose(0, 2, 1).reshape(
        NNZ, NPB, NPB)


# speedup vs baseline: 64.0734x; 1.4896x over previous
import functools

import jax
import jax.numpy as jnp
from jax import lax
from jax.experimental import pallas as pl
from jax.experimental.pallas import tpu as pltpu
from jax.experimental.pallas import tpu_sc as plsc

NUM_BLOCKS = 1024
NPB = 4
B = 16
T = 32
NNZ = 32768
N = NUM_BLOCKS * NPB
HEBB_LR = 0.01
OJA_DECAY = 0.001
MAX_NORM = 1.0
DELTA_MAX_NORM = 0.1
EPS = 1e-6

NW = 32
EPW = NNZ // NW
BLK = NPB * NPB
CHUNK = 128
NCHUNK = EPW // CHUNK
NGRP = CHUNK // 16


def _prep_body(s_ref, e_ref, v_ref, inv_ref, q_ref, sd_ref, h_ref):
    s = s_ref[0]
    e = e_ref[0]
    v = v_ref[0]
    inv = inv_ref[0]
    q_ref[0] = (v * (1.0 - s * s) + 0.5 * e) * (1.0 / B)
    sd_ref[0] = s * inv
    h_ref[0, 0] = jnp.sum(s * s, axis=0) * (OJA_DECAY / B)


def _prep(s_traj, e_traj, v_traj, inv_traj):
    return pl.pallas_call(
        _prep_body,
        grid=(T,),
        in_specs=[
            pl.BlockSpec((1, B, N), lambda t: (t, 0, 0)),
            pl.BlockSpec((1, B, N), lambda t: (t, 0, 0)),
            pl.BlockSpec((1, B, N), lambda t: (t, 0, 0)),
            pl.BlockSpec((1, B, 1), lambda t: (t, 0, 0)),
        ],
        out_specs=[
            pl.BlockSpec((1, B, N), lambda t: (t, 0, 0)),
            pl.BlockSpec((1, B, N), lambda t: (t, 0, 0)),
            pl.BlockSpec((1, 1, N), lambda t: (t, 0, 0)),
        ],
        out_shape=[
            jax.ShapeDtypeStruct((T, B, N), jnp.float32),
            jax.ShapeDtypeStruct((T, B, N), jnp.float32),
            jax.ShapeDtypeStruct((T, 1, N), jnp.float32),
        ],
    )(s_traj, e_traj, v_traj, inv_traj)


def _rsqrt(x):
    xi = lax.bitcast_convert_type(x, jnp.int32)
    yi = jnp.int32(0x5F3759DF) - lax.shift_right_logical(xi, 1)
    y = lax.bitcast_convert_type(yi, jnp.float32)
    xh = 0.5 * x
    for _ in range(3):
        y = y * (1.5 - xh * y * y)
    return y


ROWW = 80

_MESH = plsc.VectorSubcoreMesh(core_axis_name="c", subcore_axis_name="s")


@functools.partial(
    pl.kernel,
    mesh=_MESH,
    compiler_params=pltpu.CompilerParams(needs_layout_passes=False,
                                         use_tc_tiling_on_sc=False),
    out_type=jax.ShapeDtypeStruct((NW * BLK * EPW,), jnp.float32),
    scratch_types=[
        pltpu.VMEM((EPW,), jnp.int32),
        pltpu.VMEM((EPW,), jnp.int32),
        pltpu.VMEM((BLK * EPW,), jnp.float32),
        pltpu.VMEM((N,), jnp.float32),
        pltpu.VMEM((NUM_BLOCKS,), jnp.float32),
        pltpu.VMEM((CHUNK,), jnp.int32),
        pltpu.VMEM((CHUNK,), jnp.int32),
        pltpu.VMEM((CHUNK, ROWW), jnp.float32),
        pltpu.VMEM((CHUNK, ROWW), jnp.float32),
        pltpu.SemaphoreType.DMA,
        pltpu.VMEM((CHUNK,), jnp.int32),
        pltpu.VMEM((CHUNK,), jnp.int32),
        pltpu.VMEM((CHUNK, ROWW), jnp.float32),
        pltpu.VMEM((CHUNK, ROWW), jnp.float32),
        pltpu.SemaphoreType.DMA,
    ],
)
def _sc_update(q_hbm, s_hbm, h_hbm, rows_hbm, cols_hbm, w_hbm, w_out,
               rows_v, cols_v, w_v, h_v, g_v,
               idxr0, idxc0, qr0, sr0, sem0,
               idxr1, idxc1, qr1, sr1, sem1):
    wid = lax.axis_index("s") * 2 + lax.axis_index("c")
    ebase = wid * EPW
    wbase = wid * (BLK * EPW)
    pltpu.sync_copy(rows_hbm.at[pl.ds(ebase, EPW)], rows_v)
    pltpu.sync_copy(cols_hbm.at[pl.ds(ebase, EPW)], cols_v)
    pltpu.sync_copy(w_hbm.at[pl.ds(wbase, BLK * EPW)], w_v)

    lanes = lax.iota(jnp.int32, 16)
    nsteps = T * NCHUNK

    lshift = lanes * (T * NUM_BLOCKS)

    def fire(s, idxr, idxc, qr, sr, sem):
        s = jnp.minimum(s, nsteps - 1)
        toff = (s // NCHUNK) * NUM_BLOCKS
        cbase = (s % NCHUNK) * CHUNK
        for j in range(CHUNK // 16):
            rseg = rows_v[pl.ds(cbase + j * 16, 16)]
            cseg = cols_v[pl.ds(cbase + j * 16, 16)]
            idxr[pl.ds(j * 16, 16)] = rseg + toff + lshift
            idxc[pl.ds(j * 16, 16)] = cseg + toff + lshift
        pltpu.async_copy(q_hbm.at[idxr], qr, sem)
        pltpu.async_copy(s_hbm.at[idxc], sr, sem)

    def wait_pair(idxr, idxc, qr, sr, sem):
        pltpu.make_async_copy(q_hbm.at[idxr], qr, sem).wait()
        pltpu.make_async_copy(s_hbm.at[idxc], sr, sem).wait()

    def update_g(t):
        pltpu.sync_copy(h_hbm.at[pl.ds(t * N, N)], h_v)

        @plsc.parallel_loop(0, NUM_BLOCKS // 16, unroll=2)
        def g_body(j):
            base = j * 16
            bi = (lanes + base) * NPB
            acc = plsc.load_gather(h_v, [bi])
            for cc in range(1, NPB):
                acc = acc + plsc.load_gather(h_v, [bi + cc])
            g_v[pl.ds(base, 16)] = acc

    def consume(s, idxr, idxc, qr, sr, sem):
        wait_pair(idxr, idxc, qr, sr, sem)
        cbase = (s % NCHUNK) * CHUNK

        def grp_body(gi):
            e0 = gi * 16
            elanes = lanes + e0
            acc = [jnp.zeros((16,), jnp.float32) for _ in range(BLK)]
            for b in range(B):
                qa = [plsc.load_gather(qr, [elanes, lanes + (b * NPB + i)])
                      for i in range(NPB)]
                sb = [plsc.load_gather(sr, [elanes, lanes + (b * NPB + j2)])
                      for j2 in range(NPB)]
                for i in range(NPB):
                    for j2 in range(NPB):
                        acc[i * NPB + j2] = acc[i * NPB + j2] + qa[i] * sb[j2]
            off = cbase + e0
            rv = rows_v[pl.ds(off, 16)]
            gq = plsc.load_gather(g_v, [rv])
            wv = [w_v[pl.ds(ij * EPW + off, 16)] for ij in range(BLK)]
            d = [acc[ij] - gq * wv[ij] for ij in range(BLK)]
            ssd = d[0] * d[0]
            for ij in range(1, BLK):
                ssd = ssd + d[ij] * d[ij]
            scd = jnp.minimum(1.0, DELTA_MAX_NORM * _rsqrt(ssd + EPS))
            step = HEBB_LR * scd
            wn = [wv[ij] + step * d[ij] for ij in range(BLK)]
            ssw = wn[0] * wn[0]
            for ij in range(1, BLK):
                ssw = ssw + wn[ij] * wn[ij]
            scw = jnp.minimum(1.0, MAX_NORM * _rsqrt(ssw + EPS))
            for ij in range(BLK):
                w_v[pl.ds(ij * EPW + off, 16)] = wn[ij] * scw

        plsc.parallel_loop(0, NGRP, unroll=2)(grp_body)

    buf0 = (idxr0, idxc0, qr0, sr0, sem0)
    buf1 = (idxr1, idxc1, qr1, sr1, sem1)

    fire(0, *buf0)

    def pair_body(i, carry):
        s = i * 2

        @pl.when(s % NCHUNK == 0)
        def _():
            update_g(s // NCHUNK)

        fire(s + 1, *buf1)
        consume(s, *buf0)
        fire(s + 2, *buf0)
        consume(s + 1, *buf1)
        return carry

    lax.fori_loop(0, nsteps // 2, pair_body, 0)
    wait_pair(*buf0)
    pltpu.sync_copy(w_v, w_out.at[pl.ds(wbase, BLK * EPW)])


def kernel(system_states_trajectory, eligibility_traces_trajectory,
           activity_traces_trajectory, inverse_state_norms_trajectory,
           variational_gradient_trajectory, weight_values, weight_rows,
           weight_cols, trophic_support_map, activity_bias):
    del activity_traces_trajectory, trophic_support_map, activity_bias
    qd, sd, hd = _prep(system_states_trajectory,
                       eligibility_traces_trajectory,
                       variational_gradient_trajectory,
                       inverse_state_norms_trajectory)
    q_rows = qd.reshape(T, B, NUM_BLOCKS, NPB).transpose(0, 2, 1, 3).reshape(
        T * NUM_BLOCKS, B * NPB)
    s_rows = sd.reshape(T, B, NUM_BLOCKS, NPB).transpose(0, 2, 1, 3).reshape(
        T * NUM_BLOCKS, B * NPB)
    q_tbl = jnp.concatenate(
        [jnp.pad(q_rows, ((0, 0), (l, ROWW - B * NPB - l)))
         for l in range(16)], axis=0)
    s_tbl = jnp.concatenate(
        [jnp.pad(s_rows, ((0, 0), (l, ROWW - B * NPB - l)))
         for l in range(16)], axis=0)
    h_flat = hd.reshape(T * N)
    w_flat = weight_values.reshape(NW, EPW, BLK).transpose(0, 2, 1).reshape(-1)
    w_out = _sc_update(q_tbl, s_tbl, h_flat, weight_rows, weight_cols, w_flat)
    return w_out.reshape(NW, BLK, EPW).transpose(0, 2, 1).reshape(
        NNZ, NPB, NPB)


# speedup vs baseline: 70.9196x; 1.1068x over previous
import functools

import jax
import jax.numpy as jnp
from jax import lax
from jax.experimental import pallas as pl
from jax.experimental.pallas import tpu as pltpu
from jax.experimental.pallas import tpu_sc as plsc

NUM_BLOCKS = 1024
NPB = 4
B = 16
T = 32
NNZ = 32768
N = NUM_BLOCKS * NPB
HEBB_LR = 0.01
OJA_DECAY = 0.001
MAX_NORM = 1.0
DELTA_MAX_NORM = 0.1
EPS = 1e-6

NW = 32
EPW = NNZ // NW
BLK = NPB * NPB
CHUNK = 128
NCHUNK = EPW // CHUNK
NGRP = CHUNK // 16


def _prep_body(s_ref, e_ref, v_ref, inv_ref, q_ref, sd_ref, h_ref):
    s = s_ref[0]
    e = e_ref[0]
    v = v_ref[0]
    inv = inv_ref[0]
    q_ref[0] = (v * (1.0 - s * s) + 0.5 * e) * (1.0 / B)
    sd_ref[0] = s * inv
    h_ref[0, 0] = jnp.sum(s * s, axis=0) * (OJA_DECAY / B)


def _prep(s_traj, e_traj, v_traj, inv_traj):
    return pl.pallas_call(
        _prep_body,
        grid=(T,),
        in_specs=[
            pl.BlockSpec((1, B, N), lambda t: (t, 0, 0)),
            pl.BlockSpec((1, B, N), lambda t: (t, 0, 0)),
            pl.BlockSpec((1, B, N), lambda t: (t, 0, 0)),
            pl.BlockSpec((1, B, 1), lambda t: (t, 0, 0)),
        ],
        out_specs=[
            pl.BlockSpec((1, B, N), lambda t: (t, 0, 0)),
            pl.BlockSpec((1, B, N), lambda t: (t, 0, 0)),
            pl.BlockSpec((1, 1, N), lambda t: (t, 0, 0)),
        ],
        out_shape=[
            jax.ShapeDtypeStruct((T, B, N), jnp.float32),
            jax.ShapeDtypeStruct((T, B, N), jnp.float32),
            jax.ShapeDtypeStruct((T, 1, N), jnp.float32),
        ],
    )(s_traj, e_traj, v_traj, inv_traj)


_RCHUNK = 4096


def _shift_body(q_ref, s_ref, qo_ref, so_ref):
    l = pl.program_id(0)
    j = pl.program_id(1)
    c = lax.broadcasted_iota(jnp.int32, (B * NPB, ROWW), 0)
    q = lax.broadcasted_iota(jnp.int32, (B * NPB, ROWW), 1)
    shift = jnp.where(q == c + l, 1.0, 0.0)
    rows = pl.ds(j * _RCHUNK, _RCHUNK)
    qo_ref[0] = jnp.dot(q_ref[rows, :], shift,
                        preferred_element_type=jnp.float32)
    so_ref[0] = jnp.dot(s_ref[rows, :], shift,
                        preferred_element_type=jnp.float32)


def _shift_expand(q_rows, s_rows):
    nrows = T * NUM_BLOCKS
    return pl.pallas_call(
        _shift_body,
        grid=(16, nrows // _RCHUNK),
        in_specs=[
            pl.BlockSpec((nrows, B * NPB), lambda l, j: (0, 0)),
            pl.BlockSpec((nrows, B * NPB), lambda l, j: (0, 0)),
        ],
        out_specs=[
            pl.BlockSpec((1, _RCHUNK, ROWW), lambda l, j: (l, j, 0)),
            pl.BlockSpec((1, _RCHUNK, ROWW), lambda l, j: (l, j, 0)),
        ],
        out_shape=[
            jax.ShapeDtypeStruct((16, nrows, ROWW), jnp.float32),
            jax.ShapeDtypeStruct((16, nrows, ROWW), jnp.float32),
        ],
    )(q_rows, s_rows)


def _rsqrt(x):
    xi = lax.bitcast_convert_type(x, jnp.int32)
    yi = jnp.int32(0x5F3759DF) - lax.shift_right_logical(xi, 1)
    y = lax.bitcast_convert_type(yi, jnp.float32)
    xh = 0.5 * x
    for _ in range(3):
        y = y * (1.5 - xh * y * y)
    return y


ROWW = 80

_MESH = plsc.VectorSubcoreMesh(core_axis_name="c", subcore_axis_name="s")


@functools.partial(
    pl.kernel,
    mesh=_MESH,
    compiler_params=pltpu.CompilerParams(needs_layout_passes=False,
                                         use_tc_tiling_on_sc=False),
    out_type=jax.ShapeDtypeStruct((NW * BLK * EPW,), jnp.float32),
    scratch_types=[
        pltpu.VMEM((EPW,), jnp.int32),
        pltpu.VMEM((EPW,), jnp.int32),
        pltpu.VMEM((BLK * EPW,), jnp.float32),
        pltpu.VMEM((N,), jnp.float32),
        pltpu.VMEM((NUM_BLOCKS,), jnp.float32),
        pltpu.VMEM((CHUNK,), jnp.int32),
        pltpu.VMEM((CHUNK,), jnp.int32),
        pltpu.VMEM((CHUNK, ROWW), jnp.float32),
        pltpu.VMEM((CHUNK, ROWW), jnp.float32),
        pltpu.SemaphoreType.DMA,
        pltpu.VMEM((CHUNK,), jnp.int32),
        pltpu.VMEM((CHUNK,), jnp.int32),
        pltpu.VMEM((CHUNK, ROWW), jnp.float32),
        pltpu.VMEM((CHUNK, ROWW), jnp.float32),
        pltpu.SemaphoreType.DMA,
    ],
)
def _sc_update(q_hbm, s_hbm, h_hbm, rows_hbm, cols_hbm, w_hbm, w_out,
               rows_v, cols_v, w_v, h_v, g_v,
               idxr0, idxc0, qr0, sr0, sem0,
               idxr1, idxc1, qr1, sr1, sem1):
    wid = lax.axis_index("s") * 2 + lax.axis_index("c")
    ebase = wid * EPW
    wbase = wid * (BLK * EPW)
    pltpu.sync_copy(rows_hbm.at[pl.ds(ebase, EPW)], rows_v)
    pltpu.sync_copy(cols_hbm.at[pl.ds(ebase, EPW)], cols_v)
    pltpu.sync_copy(w_hbm.at[pl.ds(wbase, BLK * EPW)], w_v)

    lanes = lax.iota(jnp.int32, 16)
    nsteps = T * NCHUNK

    lshift = lanes * (T * NUM_BLOCKS)

    def fire(s, idxr, idxc, qr, sr, sem):
        s = jnp.minimum(s, nsteps - 1)
        toff = (s // NCHUNK) * NUM_BLOCKS
        cbase = (s % NCHUNK) * CHUNK
        for j in range(CHUNK // 16):
            rseg = rows_v[pl.ds(cbase + j * 16, 16)]
            cseg = cols_v[pl.ds(cbase + j * 16, 16)]
            idxr[pl.ds(j * 16, 16)] = rseg + toff + lshift
            idxc[pl.ds(j * 16, 16)] = cseg + toff + lshift
        pltpu.async_copy(q_hbm.at[idxr], qr, sem)
        pltpu.async_copy(s_hbm.at[idxc], sr, sem)

    def wait_pair(idxr, idxc, qr, sr, sem):
        pltpu.make_async_copy(q_hbm.at[idxr], qr, sem).wait()
        pltpu.make_async_copy(s_hbm.at[idxc], sr, sem).wait()

    def update_g(t):
        pltpu.sync_copy(h_hbm.at[pl.ds(t * N, N)], h_v)

        @plsc.parallel_loop(0, NUM_BLOCKS // 16, unroll=2)
        def g_body(j):
            base = j * 16
            bi = (lanes + base) * NPB
            acc = plsc.load_gather(h_v, [bi])
            for cc in range(1, NPB):
                acc = acc + plsc.load_gather(h_v, [bi + cc])
            g_v[pl.ds(base, 16)] = acc

    def consume(s, idxr, idxc, qr, sr, sem):
        wait_pair(idxr, idxc, qr, sr, sem)
        cbase = (s % NCHUNK) * CHUNK

        def grp_body(gi):
            e0 = gi * 16
            elanes = lanes + e0
            acc = [jnp.zeros((16,), jnp.float32) for _ in range(BLK)]
            for b in range(B):
                qa = [plsc.load_gather(qr, [elanes, lanes + (b * NPB + i)])
                      for i in range(NPB)]
                sb = [plsc.load_gather(sr, [elanes, lanes + (b * NPB + j2)])
                      for j2 in range(NPB)]
                for i in range(NPB):
                    for j2 in range(NPB):
                        acc[i * NPB + j2] = acc[i * NPB + j2] + qa[i] * sb[j2]
            off = cbase + e0
            rv = rows_v[pl.ds(off, 16)]
            gq = plsc.load_gather(g_v, [rv])
            wv = [w_v[pl.ds(ij * EPW + off, 16)] for ij in range(BLK)]
            d = [acc[ij] - gq * wv[ij] for ij in range(BLK)]
            ssd = d[0] * d[0]
            for ij in range(1, BLK):
                ssd = ssd + d[ij] * d[ij]
            scd = jnp.minimum(1.0, DELTA_MAX_NORM * _rsqrt(ssd + EPS))
            step = HEBB_LR * scd
            wn = [wv[ij] + step * d[ij] for ij in range(BLK)]
            ssw = wn[0] * wn[0]
            for ij in range(1, BLK):
                ssw = ssw + wn[ij] * wn[ij]
            scw = jnp.minimum(1.0, MAX_NORM * _rsqrt(ssw + EPS))
            for ij in range(BLK):
                w_v[pl.ds(ij * EPW + off, 16)] = wn[ij] * scw

        plsc.parallel_loop(0, NGRP, unroll=2)(grp_body)

    buf0 = (idxr0, idxc0, qr0, sr0, sem0)
    buf1 = (idxr1, idxc1, qr1, sr1, sem1)

    fire(0, *buf0)

    def pair_body(i, carry):
        s = i * 2

        @pl.when(s % NCHUNK == 0)
        def _():
            update_g(s // NCHUNK)

        fire(s + 1, *buf1)
        consume(s, *buf0)
        fire(s + 2, *buf0)
        consume(s + 1, *buf1)
        return carry

    lax.fori_loop(0, nsteps // 2, pair_body, 0)
    wait_pair(*buf0)
    pltpu.sync_copy(w_v, w_out.at[pl.ds(wbase, BLK * EPW)])


def kernel(system_states_trajectory, eligibility_traces_trajectory,
           activity_traces_trajectory, inverse_state_norms_trajectory,
           variational_gradient_trajectory, weight_values, weight_rows,
           weight_cols, trophic_support_map, activity_bias):
    del activity_traces_trajectory, trophic_support_map, activity_bias
    qd, sd, hd = _prep(system_states_trajectory,
                       eligibility_traces_trajectory,
                       variational_gradient_trajectory,
                       inverse_state_norms_trajectory)
    q_rows = qd.reshape(T, B, NUM_BLOCKS, NPB).transpose(0, 2, 1, 3).reshape(
        T * NUM_BLOCKS, B * NPB)
    s_rows = sd.reshape(T, B, NUM_BLOCKS, NPB).transpose(0, 2, 1, 3).reshape(
        T * NUM_BLOCKS, B * NPB)
    q_tbl, s_tbl = _shift_expand(q_rows, s_rows)
    q_tbl = q_tbl.reshape(16 * T * NUM_BLOCKS, ROWW)
    s_tbl = s_tbl.reshape(16 * T * NUM_BLOCKS, ROWW)
    h_flat = hd.reshape(T * N)
    w_flat = weight_values.reshape(NW, EPW, BLK).transpose(0, 2, 1).reshape(-1)
    w_out = _sc_update(q_tbl, s_tbl, h_flat, weight_rows, weight_cols, w_flat)
    return w_out.reshape(NW, BLK, EPW).transpose(0, 2, 1).reshape(
        NNZ, NPB, NPB)
